# Initial kernel scaffold; baseline (speedup 1.0000x reference)
import jax
import jax.numpy as jnp
from jax.experimental import pallas as pl


def kernel(input_ids, weight):
    raise NotImplementedError("write your pallas kernel here")


# speedup vs baseline: 6.3491x; 6.3491x over previous
import functools

import jax
import jax.numpy as jnp
from jax import lax
from jax.experimental import pallas as pl
from jax.experimental.pallas import tpu as pltpu
from jax.experimental.pallas import tpu_sc as plsc

_BATCH = 4096
_SEQ = 50
_DIM = 128
_N = _BATCH * _SEQ
_NW = 32
_ROWS_PER_W = _N // _NW
_CHUNK = 128
_NCHUNK = _ROWS_PER_W // _CHUNK


def _sc_gather(idx_flat, weight):
    mesh = plsc.VectorSubcoreMesh(core_axis_name="c", subcore_axis_name="s")

    @functools.partial(
        pl.kernel,
        mesh=mesh,
        out_type=jax.ShapeDtypeStruct((_N, _DIM), jnp.float32),
        scratch_types=[
            pltpu.VMEM((_CHUNK,), jnp.int32),
            pltpu.VMEM((_CHUNK, _DIM), jnp.float32),
            pltpu.SemaphoreType.DMA,
        ],
    )
    def k(idx_hbm, table_hbm, out_hbm, idx_v, rows_v, sem):
        cid = lax.axis_index("c")
        sid = lax.axis_index("s")
        wid = sid * 2 + cid
        base = wid * _ROWS_PER_W

        def body(i, carry):
            off = base + i * _CHUNK
            pltpu.sync_copy(idx_hbm.at[pl.ds(off, _CHUNK)], idx_v)
            pltpu.async_copy(table_hbm.at[idx_v], rows_v, sem).wait()
            pltpu.sync_copy(rows_v, out_hbm.at[pl.ds(off, _CHUNK)])
            return carry

        lax.fori_loop(0, _NCHUNK, body, 0)

    return k(idx_flat, weight)


def kernel(input_ids, weight):
    idx_flat = jnp.transpose(input_ids).reshape(_N).astype(jnp.int32)
    out_flat = _sc_gather(idx_flat, weight)
    return out_flat.reshape(_SEQ, _BATCH, _DIM)


# speedup vs baseline: 10.3925x; 1.6368x over previous
import functools

import jax
import jax.numpy as jnp
from jax import lax
from jax.experimental import pallas as pl
from jax.experimental.pallas import tpu as pltpu
from jax.experimental.pallas import tpu_sc as plsc

_BATCH = 4096
_SEQ = 50
_DIM = 128
_N = _BATCH * _SEQ
_NW = 32
_ROWS_PER_W = _N // _NW
_CHUNK = 128
_NCHUNK = _ROWS_PER_W // _CHUNK
_NPAIR = _NCHUNK // 2


def _sc_gather(idx_flat, weight):
    mesh = plsc.VectorSubcoreMesh(core_axis_name="c", subcore_axis_name="s")

    @functools.partial(
        pl.kernel,
        mesh=mesh,
        out_type=jax.ShapeDtypeStruct((_N, _DIM), jnp.float32),
        scratch_types=[
            pltpu.VMEM((_ROWS_PER_W,), jnp.int32),
            pltpu.VMEM((_CHUNK, _DIM), jnp.float32),
            pltpu.VMEM((_CHUNK, _DIM), jnp.float32),
            pltpu.SemaphoreType.DMA,
            pltpu.SemaphoreType.DMA,
            pltpu.SemaphoreType.DMA,
            pltpu.SemaphoreType.DMA,
        ],
    )
    def k(idx_hbm, table_hbm, out_hbm, idx_v, r0, r1, g0, g1, o0, o1):
        cid = lax.axis_index("c")
        sid = lax.axis_index("s")
        wid = sid * 2 + cid
        base = wid * _ROWS_PER_W
        pltpu.sync_copy(idx_hbm.at[pl.ds(base, _ROWS_PER_W)], idx_v)

        def idxs(c):
            return idx_v.at[pl.ds(c * _CHUNK, _CHUNK)]

        def out_at(c):
            return out_hbm.at[pl.ds(base + c * _CHUNK, _CHUNK)]

        def start_gather(c, rv, sem):
            pltpu.make_async_copy(table_hbm.at[idxs(c)], rv, sem).start()

        def wait_gather(c, rv, sem):
            pltpu.make_async_copy(table_hbm.at[idxs(c)], rv, sem).wait()

        def start_out(c, rv, sem):
            pltpu.make_async_copy(rv, out_at(c), sem).start()

        def wait_out(c, rv, sem):
            pltpu.make_async_copy(rv, out_at(c), sem).wait()

        def pair(g, first, last):
            a = 2 * g
            if not first:
                wait_out(a - 1, r1, o1)
            start_gather(a + 1, r1, g1)
            wait_gather(a, r0, g0)
            start_out(a, r0, o0)
            wait_out(a, r0, o0)
            if not last:
                start_gather(a + 2, r0, g0)
            wait_gather(a + 1, r1, g1)
            start_out(a + 1, r1, o1)

        start_gather(0, r0, g0)
        pair(0, first=True, last=False)

        def body(g, carry):
            pair(g, first=False, last=False)
            return carry

        lax.fori_loop(1, _NPAIR - 1, body, 0)
        pair(_NPAIR - 1, first=False, last=True)
        wait_out(_NCHUNK - 1, r1, o1)

    return k(idx_flat, weight)


def kernel(input_ids, weight):
    idx_flat = jnp.transpose(input_ids).reshape(_N).astype(jnp.int32)
    out_flat = _sc_gather(idx_flat, weight)
    return out_flat.reshape(_SEQ, _BATCH, _DIM)


# speedup vs baseline: 10.4319x; 1.0038x over previous
import functools

import jax
import jax.numpy as jnp
from jax import lax
from jax.experimental import pallas as pl
from jax.experimental.pallas import tpu as pltpu
from jax.experimental.pallas import tpu_sc as plsc

_BATCH = 4096
_SEQ = 50
_DIM = 128
_N = _BATCH * _SEQ
_NW = 32
_ROWS_PER_W = _N // _NW
_CHUNK = 320
_NCHUNK = _ROWS_PER_W // _CHUNK
_NPAIR = _NCHUNK // 2


def _sc_gather(idx_flat, weight):
    mesh = plsc.VectorSubcoreMesh(core_axis_name="c", subcore_axis_name="s")

    @functools.partial(
        pl.kernel,
        mesh=mesh,
        out_type=jax.ShapeDtypeStruct((_N, _DIM), jnp.float32),
        scratch_types=[
            pltpu.VMEM((_ROWS_PER_W,), jnp.int32),
            pltpu.VMEM((_CHUNK, _DIM), jnp.float32),
            pltpu.VMEM((_CHUNK, _DIM), jnp.float32),
            pltpu.SemaphoreType.DMA,
            pltpu.SemaphoreType.DMA,
            pltpu.SemaphoreType.DMA,
            pltpu.SemaphoreType.DMA,
        ],
    )
    def k(idx_hbm, table_hbm, out_hbm, idx_v, r0, r1, g0, g1, o0, o1):
        cid = lax.axis_index("c")
        sid = lax.axis_index("s")
        wid = sid * 2 + cid
        base = wid * _ROWS_PER_W
        pltpu.sync_copy(idx_hbm.at[pl.ds(base, _ROWS_PER_W)], idx_v)

        def idxs(c):
            return idx_v.at[pl.ds(c * _CHUNK, _CHUNK)]

        def out_at(c):
            return out_hbm.at[pl.ds(base + c * _CHUNK, _CHUNK)]

        def start_gather(c, rv, sem):
            pltpu.make_async_copy(table_hbm.at[idxs(c)], rv, sem).start()

        def wait_gather(c, rv, sem):
            pltpu.make_async_copy(table_hbm.at[idxs(c)], rv, sem).wait()

        def start_out(c, rv, sem):
            pltpu.make_async_copy(rv, out_at(c), sem).start()

        def wait_out(c, rv, sem):
            pltpu.make_async_copy(rv, out_at(c), sem).wait()

        def pair(g, first, last):
            a = 2 * g
            if not first:
                wait_out(a - 1, r1, o1)
            start_gather(a + 1, r1, g1)
            wait_gather(a, r0, g0)
            start_out(a, r0, o0)
            wait_out(a, r0, o0)
            if not last:
                start_gather(a + 2, r0, g0)
            wait_gather(a + 1, r1, g1)
            start_out(a + 1, r1, o1)

        start_gather(0, r0, g0)
        pair(0, first=True, last=False)

        def body(g, carry):
            pair(g, first=False, last=False)
            return carry

        lax.fori_loop(1, _NPAIR - 1, body, 0)
        pair(_NPAIR - 1, first=False, last=True)
        wait_out(_NCHUNK - 1, r1, o1)

    return k(idx_flat, weight)


def kernel(input_ids, weight):
    idx_flat = jnp.transpose(input_ids).reshape(_N).astype(jnp.int32)
    out_flat = _sc_gather(idx_flat, weight)
    return out_flat.reshape(_SEQ, _BATCH, _DIM)


# speedup vs baseline: 10.4678x; 1.0034x over previous
import functools

import jax
import jax.numpy as jnp
from jax import lax
from jax.experimental import pallas as pl
from jax.experimental.pallas import tpu as pltpu
from jax.experimental.pallas import tpu_sc as plsc

_BATCH = 4096
_SEQ = 50
_DIM = 128
_N = _BATCH * _SEQ
_NW = 32
_ROWS_PER_W = _N // _NW
_CHUNK = 160
_NCHUNK = _ROWS_PER_W // _CHUNK
_NBUF = 4


def _sc_gather(idx_flat, weight):
    mesh = plsc.VectorSubcoreMesh(core_axis_name="c", subcore_axis_name="s")

    @functools.partial(
        pl.kernel,
        mesh=mesh,
        out_type=jax.ShapeDtypeStruct((_N, _DIM), jnp.float32),
        scratch_types=[
            pltpu.VMEM((_ROWS_PER_W,), jnp.int32),
            pltpu.VMEM((_NBUF, _CHUNK, _DIM), jnp.float32),
            pltpu.SemaphoreType.DMA((_NBUF,)),
            pltpu.SemaphoreType.DMA((_NBUF,)),
        ],
    )
    def k(idx_hbm, table_hbm, out_hbm, idx_v, rows_v, gsem, osem):
        cid = lax.axis_index("c")
        sid = lax.axis_index("s")
        wid = sid * 2 + cid
        base = wid * _ROWS_PER_W
        pltpu.sync_copy(idx_hbm.at[pl.ds(base, _ROWS_PER_W)], idx_v)

        def start_gather(c, b):
            pltpu.make_async_copy(
                table_hbm.at[idx_v.at[pl.ds(c * _CHUNK, _CHUNK)]],
                rows_v.at[b], gsem.at[b]).start()

        def wait_gather(c, b):
            pltpu.make_async_copy(
                table_hbm.at[idx_v.at[pl.ds(c * _CHUNK, _CHUNK)]],
                rows_v.at[b], gsem.at[b]).wait()

        def start_out(c, b):
            pltpu.make_async_copy(
                rows_v.at[b], out_hbm.at[pl.ds(base + c * _CHUNK, _CHUNK)],
                osem.at[b]).start()

        def wait_out(c, b):
            pltpu.make_async_copy(
                rows_v.at[b], out_hbm.at[pl.ds(base + c * _CHUNK, _CHUNK)],
                osem.at[b]).wait()

        start_gather(0, 0)
        for b in range(1, _NBUF):
            start_gather(b, b)
            wait_gather(b - 1, b - 1)
            start_out(b - 1, b - 1)

        def group(g, carry):
            for b in range(_NBUF):
                c = g * _NBUF + b
                wait_out(c - _NBUF, b)
                start_gather(c, b)
                pb = (b - 1) % _NBUF
                wait_gather(c - 1, pb)
                start_out(c - 1, pb)
            return carry

        lax.fori_loop(1, _NCHUNK // _NBUF, group, 0)

        last = _NCHUNK - 1
        wait_gather(last, last % _NBUF)
        start_out(last, last % _NBUF)
        for c in range(_NCHUNK - _NBUF, _NCHUNK):
            wait_out(c, c % _NBUF)

    return k(idx_flat, weight)


def kernel(input_ids, weight):
    idx_flat = jnp.transpose(input_ids).reshape(_N).astype(jnp.int32)
    out_flat = _sc_gather(idx_flat, weight)
    return out_flat.reshape(_SEQ, _BATCH, _DIM)


# speedup vs baseline: 10.5563x; 1.0085x over previous
import functools

import jax
import jax.numpy as jnp
from jax import lax
from jax.experimental import pallas as pl
from jax.experimental.pallas import tpu as pltpu
from jax.experimental.pallas import tpu_sc as plsc

_BATCH = 4096
_SEQ = 50
_DIM = 128
_N = _BATCH * _SEQ
_NW = 32
_NSUB = 16
_ROWS_PER_W = _N // _NW
_CHUNK = 160
_NCHUNK = _ROWS_PER_W // _CHUNK
_NBUF = 4
_NSP = 2


def _sc_gather(idx_flat, weight):
    mesh = plsc.VectorSubcoreMesh(core_axis_name="c", subcore_axis_name="s")

    @functools.partial(
        pl.kernel,
        mesh=mesh,
        out_type=jax.ShapeDtypeStruct((_N, _DIM), jnp.float32),
        scratch_types=[
            pltpu.VMEM((_ROWS_PER_W,), jnp.int32),
            pltpu.VMEM((_NBUF, _CHUNK, _DIM), jnp.float32),
            pltpu.VMEM_SHARED((_NSUB, _NSP, _CHUNK, _DIM), jnp.float32),
            pltpu.SemaphoreType.DMA((_NBUF,)),
            pltpu.SemaphoreType.DMA((_NSP,)),
        ],
    )
    def k(idx_hbm, table_hbm, out_hbm, idx_v, rows_v, sp, gsem, ysem):
        cid = lax.axis_index("c")
        sid = lax.axis_index("s")
        wid = sid * 2 + cid
        base = wid * _ROWS_PER_W
        pltpu.sync_copy(idx_hbm.at[pl.ds(base, _ROWS_PER_W)], idx_v)

        def start_gather(c, b):
            pltpu.make_async_copy(
                table_hbm.at[idx_v.at[pl.ds(c * _CHUNK, _CHUNK)]],
                rows_v.at[b], gsem.at[b]).start()

        def wait_gather(c, b):
            pltpu.make_async_copy(
                table_hbm.at[idx_v.at[pl.ds(c * _CHUNK, _CHUNK)]],
                rows_v.at[b], gsem.at[b]).wait()

        def stage(c):
            b = c % _NBUF
            m = c % _NSP
            pltpu.sync_copy(rows_v.at[b], sp.at[sid, m])
            pltpu.make_async_copy(
                sp.at[sid, m],
                out_hbm.at[pl.ds(base + c * _CHUNK, _CHUNK)],
                ysem.at[m]).start()

        def wait_out(c):
            m = c % _NSP
            pltpu.make_async_copy(
                sp.at[sid, m],
                out_hbm.at[pl.ds(base + c * _CHUNK, _CHUNK)],
                ysem.at[m]).wait()

        start_gather(0, 0)
        for b in range(1, _NBUF):
            start_gather(b, b)
            wait_gather(b - 1, b - 1)
            if b - 1 >= _NSP:
                wait_out(b - 1 - _NSP)
            stage(b - 1)

        def group(g, carry):
            for b in range(_NBUF):
                c = g * _NBUF + b
                start_gather(c, b)
                wait_gather(c - 1, (b - 1) % _NBUF)
                wait_out(c - 1 - _NSP)
                stage(c - 1)
            return carry

        lax.fori_loop(1, _NCHUNK // _NBUF, group, 0)

        last = _NCHUNK - 1
        wait_gather(last, last % _NBUF)
        wait_out(last - _NSP)
        stage(last)
        for c in range(_NCHUNK - _NSP, _NCHUNK):
            wait_out(c)

    return k(idx_flat, weight)


def kernel(input_ids, weight):
    idx_flat = jnp.transpose(input_ids).reshape(_N).astype(jnp.int32)
    out_flat = _sc_gather(idx_flat, weight)
    return out_flat.reshape(_SEQ, _BATCH, _DIM)
